# baseline (speedup 1.0000x reference)
import functools

import jax
import jax.numpy as jnp
from jax import lax
from jax.experimental import pallas as pl
from jax.experimental.pallas import tpu as pltpu
from jax.experimental.pallas import tpu_sc as plsc

NC = 2
NS = 16
L = 16
NW = NC * NS

B, M, D = 4096, 200, 64
N = B * M
PER_W = N // NW
C = 2 * M
NCHUNK = PER_W // C
SUB = 80
NSUB = C // SUB


def _sc_embed(x_flat, token_table, pos_ext):
    mesh = plsc.VectorSubcoreMesh(core_axis_name="c", subcore_axis_name="s")

    @functools.partial(
        pl.kernel,
        out_type=jax.ShapeDtypeStruct((N, D), jnp.float32),
        mesh=mesh,
        scratch_types=[
            pltpu.VMEM((C,), jnp.int32),
            pltpu.VMEM((C, D), jnp.float32),
            pltpu.VMEM((C, D), jnp.float32),
            pltpu.SemaphoreType.DMA,
        ],
        compiler_params=pltpu.CompilerParams(use_tc_tiling_on_sc=False),
    )
    def body(tok_hbm, idx_hbm, pos_hbm, out_hbm, idx_v, rows_v, pos_v, gsem):
        wid = lax.axis_index("s") * NC + lax.axis_index("c")
        base = wid * PER_W
        pltpu.sync_copy(pos_hbm, pos_v)

        def chunk_body(g, carry):
            row0 = base + g * C
            pltpu.sync_copy(idx_hbm.at[pl.ds(row0, C)], idx_v)
            cps = [
                pltpu.async_copy(
                    tok_hbm.at[idx_v.at[pl.ds(j * SUB, SUB)]],
                    rows_v.at[pl.ds(j * SUB, SUB)],
                    gsem,
                )
                for j in range(NSUB)
            ]
            for cp in cps:
                cp.wait()

            def row_body(r, c2):
                for c4 in range(D // L):
                    sl = pl.ds(c4 * L, L)
                    rows_v[r, sl] = rows_v[r, sl] + pos_v[r, sl]
                return c2

            lax.fori_loop(0, C, row_body, 0)
            pltpu.sync_copy(rows_v, out_hbm.at[pl.ds(row0, C)])
            return carry

        lax.fori_loop(0, NCHUNK, chunk_body, 0)

    return body(token_table, x_flat, pos_ext)


def kernel(x, token_table, pos_table):
    x_flat = x.reshape(-1).astype(jnp.int32)
    pos_ext = jnp.concatenate([pos_table, pos_table], axis=0)
    out = _sc_embed(x_flat, token_table, pos_ext)
    return out.reshape(B, M, D)


# speedup vs baseline: 1.1119x; 1.1119x over previous
import functools

import jax
import jax.numpy as jnp
from jax import lax
from jax.experimental import pallas as pl
from jax.experimental.pallas import tpu as pltpu
from jax.experimental.pallas import tpu_sc as plsc

NC = 2
NS = 16
L = 16
NW = NC * NS

B, M, D = 4096, 200, 64
N = B * M
PER_W = N // NW
C = 2 * M
NCHUNK = PER_W // C
SUB = 80
NSUB = C // SUB


def _sc_embed(x_flat, token_table, pos_ext):
    mesh = plsc.VectorSubcoreMesh(core_axis_name="c", subcore_axis_name="s")

    @functools.partial(
        pl.kernel,
        out_type=jax.ShapeDtypeStruct((N, D), jnp.float32),
        mesh=mesh,
        scratch_types=[
            pltpu.VMEM((PER_W,), jnp.int32),
            [pltpu.VMEM((C, D), jnp.float32) for _ in range(2)],
            pltpu.VMEM((C, D), jnp.float32),
            [pltpu.SemaphoreType.DMA for _ in range(2)],
            [pltpu.SemaphoreType.DMA for _ in range(2)],
        ],
        compiler_params=pltpu.CompilerParams(use_tc_tiling_on_sc=False),
    )
    def body(tok_hbm, idx_hbm, pos_hbm, out_hbm, idx_v, rows, pos_v, gsems, osems):
        wid = lax.axis_index("s") * NC + lax.axis_index("c")
        base = wid * PER_W
        pltpu.sync_copy(pos_hbm, pos_v)
        pltpu.sync_copy(idx_hbm.at[pl.ds(base, PER_W)], idx_v)

        def fire_gathers(g, buf, sem):
            off = (g % NCHUNK) * C
            return [
                pltpu.async_copy(
                    tok_hbm.at[idx_v.at[pl.ds(off + j * SUB, SUB)]],
                    rows[buf].at[pl.ds(j * SUB, SUB)],
                    sem,
                )
                for j in range(NSUB)
            ]

        def drain_out(buf, sem):
            pltpu.make_async_copy(rows[buf], out_hbm.at[pl.ds(base, C)], sem).wait()

        for cp in fire_gathers(0, 0, gsems[0]):
            cp.wait()

        @pl.loop(0, NCHUNK // 2)
        def _(i):
            for b in range(2):
                g = 2 * i + b
                nb = 1 - b
                @pl.when(g >= 1)
                def _():
                    drain_out(nb, osems[nb])

                nxt = fire_gathers(g + 1, nb, gsems[nb])

                @pl.loop(0, C)
                def _(r):
                    for c4 in range(D // L):
                        sl = pl.ds(c4 * L, L)
                        plsc.addupdate(rows[b].at[r, sl], pos_v[r, sl])

                for cp in nxt:
                    cp.wait()
                pltpu.async_copy(
                    rows[b], out_hbm.at[pl.ds(base + g * C, C)], osems[b]
                )

        drain_out(1, osems[1])

    return body(token_table, x_flat, pos_ext)


def kernel(x, token_table, pos_table):
    x_flat = x.reshape(-1).astype(jnp.int32)
    pos_ext = jnp.concatenate([pos_table, pos_table], axis=0)
    out = _sc_embed(x_flat, token_table, pos_ext)
    return out.reshape(B, M, D)
